# baseline (speedup 1.0000x reference)
import jax
import jax.numpy as jnp
from jax.experimental import pallas as pl
from jax.experimental.pallas import tpu as pltpu

_TB = 4096


def _ac_kernel(obs_ref, table_ref, out_ref):
    tb = out_ref.shape[0]
    n_rows = table_ref.shape[0]
    obs = obs_ref[...]
    code = obs[:, 0:1] * 2 + obs[:, 1:2]
    iota = jax.lax.broadcasted_iota(jnp.int32, (tb, n_rows), 1)
    onehot = (iota == code).astype(jnp.float32)
    out_ref[...] = jnp.dot(onehot, table_ref[...],
                           preferred_element_type=jnp.float32)


def kernel(obs, table):
    B = obs.shape[0]
    head_pad = table.shape[1]
    n_actions = 4

    tb = _TB if B >= _TB else ((B + 7) // 8) * 8
    B_pad = ((B + tb - 1) // tb) * tb
    if B_pad != B:
        obs = jnp.pad(obs, ((0, B_pad - B), (0, 0)))
    grid = (B_pad // tb,)

    out = pl.pallas_call(
        _ac_kernel,
        out_shape=jax.ShapeDtypeStruct((B_pad, head_pad), jnp.float32),
        grid=grid,
        in_specs=[
            pl.BlockSpec((tb, 2), lambda i: (i, 0)),
            pl.BlockSpec((8, head_pad), lambda i: (0, 0)),
        ],
        out_specs=pl.BlockSpec((tb, head_pad), lambda i: (i, 0)),
        compiler_params=pltpu.CompilerParams(
            dimension_semantics=("parallel",)),
    )(obs, table)

    logits = out[:B, :n_actions]
    value = out[:B, n_actions]
    return logits, value


# speedup vs baseline: 44.5400x; 44.5400x over previous
import jax
import jax.numpy as jnp
from jax.experimental import pallas as pl
from jax.experimental.pallas import tpu as pltpu

_LANES = 128
_N_ACT = 4
_N_CODES = 6


def _ac_kernel(code_ref, table_ref, logits_ref, value_ref):
    c = code_ref[...]
    masks = [c == k for k in range(1, _N_CODES)]

    def sel(col):
        v = jnp.broadcast_to(table_ref[0, col], c.shape)
        for k, m in enumerate(masks, start=1):
            v = jnp.where(m, table_ref[k, col], v)
        return v

    value_ref[...] = sel(_N_ACT)
    for j in range(_N_ACT):
        logits_ref[j::_N_ACT, :] = sel(j)


def kernel(obs, table):
    B = obs.shape[0]
    R = B // _LANES
    rb = 512
    while R % rb:
        rb //= 2
    nb = R // rb

    code = obs[:, 0] * 2 + obs[:, 1]
    code2d = code.reshape(R, _LANES)

    logits4, value2d = pl.pallas_call(
        _ac_kernel,
        out_shape=(
            jax.ShapeDtypeStruct((_N_ACT * R, _LANES), jnp.float32),
            jax.ShapeDtypeStruct((R, _LANES), jnp.float32),
        ),
        grid=(nb,),
        in_specs=[
            pl.BlockSpec((rb, _LANES), lambda i: (i, 0)),
            pl.BlockSpec(memory_space=pltpu.SMEM),
        ],
        out_specs=(
            pl.BlockSpec((_N_ACT * rb, _LANES), lambda i: (i, 0)),
            pl.BlockSpec((rb, _LANES), lambda i: (i, 0)),
        ),
        compiler_params=pltpu.CompilerParams(
            dimension_semantics=("parallel",)),
    )(code2d, table)

    logits = logits4.reshape(R, _N_ACT, _LANES).transpose(0, 2, 1).reshape(B, _N_ACT)
    value = value2d.reshape(B)
    return logits, value


# speedup vs baseline: 56.2161x; 1.2621x over previous
import jax
import jax.numpy as jnp
from jax.experimental import pallas as pl
from jax.experimental.pallas import tpu as pltpu

_LANES = 128
_N_ACT = 4
_N_CODES = 6


def _ac_kernel(obs_ref, table_ref, logits_ref, value_ref):
    trust = obs_ref[0::2, :]
    risk = obs_ref[1::2, :]
    c = trust * 2 + risk
    masks = [c == k for k in range(1, _N_CODES)]

    def sel(col):
        v = jnp.broadcast_to(table_ref[0, col], c.shape)
        for k, m in enumerate(masks, start=1):
            v = jnp.where(m, table_ref[k, col], v)
        return v

    value_ref[...] = sel(_N_ACT)
    for j in range(_N_ACT):
        logits_ref[j::_N_ACT, :] = sel(j)


def kernel(obs, table):
    B = obs.shape[0]
    R = B // _LANES
    rb = 512
    while R % rb:
        rb //= 2
    nb = R // rb

    obs_rows = obs.reshape(R, _LANES, 2).transpose(0, 2, 1).reshape(2 * R, _LANES)

    logits4, value2d = pl.pallas_call(
        _ac_kernel,
        out_shape=(
            jax.ShapeDtypeStruct((_N_ACT * R, _LANES), jnp.float32),
            jax.ShapeDtypeStruct((R, _LANES), jnp.float32),
        ),
        grid=(nb,),
        in_specs=[
            pl.BlockSpec((2 * rb, _LANES), lambda i: (i, 0)),
            pl.BlockSpec(memory_space=pltpu.SMEM),
        ],
        out_specs=(
            pl.BlockSpec((_N_ACT * rb, _LANES), lambda i: (i, 0)),
            pl.BlockSpec((rb, _LANES), lambda i: (i, 0)),
        ),
        compiler_params=pltpu.CompilerParams(
            dimension_semantics=("parallel",)),
    )(obs_rows, table)

    logits = logits4.reshape(R, _N_ACT, _LANES).transpose(0, 2, 1).reshape(B, _N_ACT)
    value = value2d.reshape(B)
    return logits, value


# speedup vs baseline: 94.8956x; 1.6881x over previous
import jax
import jax.numpy as jnp
from jax.experimental import pallas as pl
from jax.experimental.pallas import tpu as pltpu

_LANES = 128
_N_ACT = 4
_N_CODES = 6


def _ac_kernel(obs_ref, table_ref, logits_ref, value_ref):
    trust = obs_ref[0::2, :]
    risk = obs_ref[1::2, :]
    c = trust * 2 + risk
    masks = [c == k for k in range(1, _N_CODES)]

    def sel(col):
        v = jnp.broadcast_to(table_ref[0, col], c.shape)
        for k, m in enumerate(masks, start=1):
            v = jnp.where(m, table_ref[k, col], v)
        return v

    value_ref[...] = sel(_N_ACT)
    for j in range(_N_ACT):
        logits_ref[j::_N_ACT, :] = sel(j)


def kernel(obs, table):
    B = obs.shape[0]
    R = B // _LANES
    rb = 2048
    while R % rb:
        rb //= 2
    nb = R // rb

    obs_rows = obs.reshape(R, _LANES, 2).transpose(0, 2, 1).reshape(2 * R, _LANES)

    logits4, value2d = pl.pallas_call(
        _ac_kernel,
        out_shape=(
            jax.ShapeDtypeStruct((_N_ACT * R, _LANES), jnp.float32),
            jax.ShapeDtypeStruct((R, _LANES), jnp.float32),
        ),
        grid=(nb,),
        in_specs=[
            pl.BlockSpec((2 * rb, _LANES), lambda i: (i, 0)),
            pl.BlockSpec(memory_space=pltpu.SMEM),
        ],
        out_specs=(
            pl.BlockSpec((_N_ACT * rb, _LANES), lambda i: (i, 0)),
            pl.BlockSpec((rb, _LANES), lambda i: (i, 0)),
        ),
        compiler_params=pltpu.CompilerParams(
            dimension_semantics=("parallel",)),
    )(obs_rows, table)

    logits = logits4.reshape(R, _N_ACT, _LANES).transpose(0, 2, 1).reshape(B, _N_ACT)
    value = value2d.reshape(B)
    return logits, value


# speedup vs baseline: 101.5226x; 1.0698x over previous
import jax
import jax.numpy as jnp
from jax.experimental import pallas as pl
from jax.experimental.pallas import tpu as pltpu

_LANES = 128
_N_ACT = 4
_N_CODES = 6


def _ac_kernel(obs_ref, table_ref, logits_ref, value_ref):
    trust = obs_ref[0::2, :]
    risk = obs_ref[1::2, :]
    c = trust * 2 + risk
    masks = [c == k for k in range(1, _N_CODES)]

    def sel(col):
        v = jnp.broadcast_to(table_ref[0, col], c.shape)
        for k, m in enumerate(masks, start=1):
            v = jnp.where(m, table_ref[k, col], v)
        return v

    value_ref[...] = sel(_N_ACT)
    for j in range(_N_ACT):
        logits_ref[j::_N_ACT, :] = sel(j)


def kernel(obs, table):
    B = obs.shape[0]
    R = B // _LANES
    rb = 4096
    while R % rb:
        rb //= 2
    nb = R // rb

    obs_rows = obs.reshape(R, _LANES, 2).transpose(0, 2, 1).reshape(2 * R, _LANES)

    logits4, value2d = pl.pallas_call(
        _ac_kernel,
        out_shape=(
            jax.ShapeDtypeStruct((_N_ACT * R, _LANES), jnp.float32),
            jax.ShapeDtypeStruct((R, _LANES), jnp.float32),
        ),
        grid=(nb,),
        in_specs=[
            pl.BlockSpec((2 * rb, _LANES), lambda i: (i, 0)),
            pl.BlockSpec(memory_space=pltpu.SMEM),
        ],
        out_specs=(
            pl.BlockSpec((_N_ACT * rb, _LANES), lambda i: (i, 0)),
            pl.BlockSpec((rb, _LANES), lambda i: (i, 0)),
        ),
        compiler_params=pltpu.CompilerParams(
            dimension_semantics=("parallel",)),
    )(obs_rows, table)

    logits = logits4.reshape(R, _N_ACT, _LANES).transpose(0, 2, 1).reshape(B, _N_ACT)
    value = value2d.reshape(B)
    return logits, value
